# Initial kernel scaffold; baseline (speedup 1.0000x reference)
import functools

import jax
import jax.numpy as jnp
import numpy as np
from jax import lax
from jax.experimental import pallas as pl
from jax.experimental.pallas import tpu as pltpu
from jax.experimental.pallas import tpu_sc as plsc

_CONV_CFG = [(1, 32, 3, (1, 1), 1, False), (32, 32, 3, (1, 1), 1, True), (32, 32, 3, (1, 1), 1, True), (32, 64, 3, (3, 1), 1, False), (64, 64, 3, (1, 1), 1, True), (64, 64, 3, (1, 1), 1, True), (64, 128, 3, (3, 3), 1, False), (128, 128, 3, (1, 1), 1, True), (128, 128, 3, (1, 1), 1, True), (128, 256, 3, (3, 2), 1, False), (256, 256, 3, (1, 1), 1, True), (256, 512, 3, (1, 1), 0, False), (512, 512, 1, (1, 1), 0, False)]
_POOL = [35709, 8928, 2232, 558, 140]
_K = 6
_DECF = [32, 16, 16, 16, 3]

_NC = 2
_NS = 16
_NW = _NC * _NS


def _round_up(a, b):
    return (a + b - 1) // b * b


@functools.partial(jax.jit, static_argnames=("deg", "n_pad", "c_rows", "fuse"))
def _spmm_sc(x, cols2d, vals, sub, *, deg, n_pad, c_rows, fuse):
    m, fw = x.shape
    C = c_rows
    G = (C * deg) // 128
    assert C * deg == G * 128 and n_pad % (C * _NW) == 0
    cpw = n_pad // (C * _NW)

    mesh = plsc.VectorSubcoreMesh(core_axis_name="c", subcore_axis_name="s",
                                  num_cores=_NC, num_subcores=_NS)
    scratch = [
        pltpu.VMEM((G, 128), jnp.int32),
        pltpu.VMEM((C * deg,), jnp.float32),
        pltpu.VMEM((C * deg, fw), jnp.float32),
        pltpu.VMEM((C, fw), jnp.float32),
    ]
    if fuse:
        scratch.append(pltpu.VMEM((C, fw), jnp.float32))
    scratch.append(pltpu.SemaphoreType.DMA)

    def body(*refs):
        if fuse:
            (x_hbm, cols_hbm, vals_hbm, sub_hbm, out_hbm,
             idx_v, vals_v, rows_v, out_v, sub_v, sem) = refs
        else:
            (x_hbm, cols_hbm, vals_hbm, out_hbm,
             idx_v, vals_v, rows_v, out_v, sem) = refs
        wid = lax.axis_index("s") * _NC + lax.axis_index("c")

        def chunk(t, carry):
            ci = wid * cpw + t
            base = ci * C
            pltpu.sync_copy(cols_hbm.at[pl.ds(ci * G, G)], idx_v)
            pltpu.sync_copy(vals_hbm.at[pl.ds(base * deg, C * deg)], vals_v)
            copies = []
            for g in range(G):
                copies.append(pltpu.async_copy(
                    x_hbm.at[idx_v.at[g]], rows_v.at[pl.ds(g * 128, 128)], sem))
            if fuse:
                pltpu.sync_copy(sub_hbm.at[pl.ds(base, C)], sub_v)
            for cp in copies:
                cp.wait()

            def row(r, carry2):
                e0 = r * deg
                ws = [vals_v[e0 + d] for d in range(deg)]
                for f in range(fw // 16):
                    sl = pl.ds(f * 16, 16)
                    acc = ws[0] * rows_v[e0, sl]
                    for d in range(1, deg):
                        acc = acc + ws[d] * rows_v[e0 + d, sl]
                    if fuse:
                        acc = 2.0 * acc - sub_v[r, sl]
                    out_v[r, sl] = acc
                return carry2

            lax.fori_loop(0, C, row, 0)
            pltpu.sync_copy(out_v, out_hbm.at[pl.ds(base, C)])
            return carry

        lax.fori_loop(0, cpw, chunk, 0)

    args = (x, cols2d, vals) + ((sub,) if fuse else ())
    return pl.kernel(
        body,
        out_type=jax.ShapeDtypeStruct((n_pad, fw), jnp.float32),
        mesh=mesh,
        scratch_types=scratch,
    )(*args)


def _spmm(g, x, n, deg, sub=None):
    m, fw_in = x.shape
    if deg == 8:
        C = 64 if n >= 4096 else 16
    else:
        C = 128
    n_pad = _round_up(n, C * _NW)
    e_pad = n_pad * deg
    cols = g["cols"]
    vals = g["vals"]
    if e_pad > cols.shape[0]:
        cols = jnp.pad(cols, (0, e_pad - cols.shape[0]))
        vals = jnp.pad(vals, (0, e_pad - vals.shape[0]))
    cols2d = cols.reshape(-1, 128)

    fw = _round_up(fw_in, 32)
    if fw != fw_in:
        x = jnp.pad(x, ((0, 0), (0, fw - fw_in)))
        if sub is not None:
            sub = jnp.pad(sub, ((0, 0), (0, fw - fw_in)))
    if sub is not None and sub.shape[0] != n_pad:
        sub = jnp.pad(sub, ((0, n_pad - sub.shape[0]), (0, 0)))

    outs = []
    for f0 in range(0, fw, 128):
        f1 = min(f0 + 128, fw)
        xs = x[:, f0:f1]
        ss = sub[:, f0:f1] if sub is not None else None
        outs.append(_spmm_sc(xs, cols2d, vals, ss, deg=deg, n_pad=n_pad,
                             c_rows=C, fuse=sub is not None))
    out = outs[0] if len(outs) == 1 else jnp.concatenate(outs, axis=1)
    return out[:n, :fw_in]


def _cheb(xm, Lg, W, b, B):
    M, bf = xm.shape
    fin = bf // B
    fo = W.shape[1]
    x0 = xm
    x1 = _spmm(Lg, x0, M, 8)
    xs = [x0, x1]
    for _ in range(2, _K):
        x2 = _spmm(Lg, x1, M, 8, sub=x0)
        xs.append(x2)
        x0, x1 = x1, x2
    Wk = W.reshape(fin, _K, fo)
    y = b
    for k in range(_K):
        y = y + xs[k].reshape(M * B, fin) @ Wk[:, k, :]
    return y.reshape(M, B * fo)


def _cheb_res_block(xm, Lg, blk, B):
    h = jax.nn.relu(_cheb(xm, Lg, blk["w1"], blk["b1"], B))
    h2 = _cheb(h, Lg, blk["w2"], blk["b2"], B)
    M = xm.shape[0]
    if "ws" in blk:
        sc = (xm.reshape(-1, xm.shape[1] // B) @ blk["ws"]).reshape(M, -1)
    else:
        sc = xm
    return jax.nn.relu(h2 + sc)


def _conv_block(x, pr, cfg):
    ci, co, k, s, p, res = cfg
    out = lax.conv_general_dilated(x, pr["w"], window_strides=s,
                                   padding=[(p, p), (p, p)],
                                   dimension_numbers=("NCHW", "OIHW", "NCHW"))
    out = out + pr["b"][None, :, None, None]
    out = pr["g"][None, :, None, None] * out + pr["be"][None, :, None, None]
    if res:
        out = out + x
    return jax.nn.relu(out)


def kernel(x, params, graphs):
    out = x
    for cfg, pr in zip(_CONV_CFG, params["convs"]):
        out = _conv_block(out, pr, cfg)
    out = out.reshape(out.shape[0], -1)
    out = out / (jnp.linalg.norm(out, axis=1, keepdims=True) + 1e-12)
    B = out.shape[0]
    h = jax.nn.relu(out @ params["fc_w"] + params["fc_b"])
    hm = h.reshape(B, _POOL[-1], _DECF[0]).transpose(1, 0, 2).reshape(_POOL[-1], B * _DECF[0])
    lap, ups, blks = graphs["L"], graphs["U"], params["blocks"]
    for lvl in range(4):
        g = ups[3 - lvl]
        n_out = _POOL[3 - lvl]
        hm = _spmm(g, hm, n_out, 3)
        hm = _cheb_res_block(hm, lap[3 - lvl], blks[lvl], B)
    hm = _cheb(hm, lap[0], params["last_w"], params["last_b"], B)
    M = hm.shape[0]
    return hm.reshape(M, B, 3).transpose(1, 0, 2).reshape(B, -1)


# speedup vs baseline: 2.4750x; 2.4750x over previous
import functools

import jax
import jax.numpy as jnp
import numpy as np
from jax import lax
from jax.experimental import pallas as pl
from jax.experimental.pallas import tpu as pltpu
from jax.experimental.pallas import tpu_sc as plsc

_CONV_CFG = [(1, 32, 3, (1, 1), 1, False), (32, 32, 3, (1, 1), 1, True), (32, 32, 3, (1, 1), 1, True), (32, 64, 3, (3, 1), 1, False), (64, 64, 3, (1, 1), 1, True), (64, 64, 3, (1, 1), 1, True), (64, 128, 3, (3, 3), 1, False), (128, 128, 3, (1, 1), 1, True), (128, 128, 3, (1, 1), 1, True), (128, 256, 3, (3, 2), 1, False), (256, 256, 3, (1, 1), 1, True), (256, 512, 3, (1, 1), 0, False), (512, 512, 1, (1, 1), 0, False)]
_POOL = [35709, 8928, 2232, 558, 140]
_K = 6
_DECF = [32, 16, 16, 16, 3]

_NC = 2
_NS = 16
_NW = _NC * _NS


def _round_up(a, b):
    return (a + b - 1) // b * b


@functools.partial(jax.jit, static_argnames=("deg", "n_pad", "c_rows", "fuse"))
def _spmm_sc(x, cols2d, vals, sub, *, deg, n_pad, c_rows, fuse):
    m, fw = x.shape
    C = c_rows
    G = (C * deg) // 128
    assert C * deg == G * 128 and n_pad % (C * _NW) == 0
    cpw = n_pad // (C * _NW)

    mesh = plsc.VectorSubcoreMesh(core_axis_name="c", subcore_axis_name="s",
                                  num_cores=_NC, num_subcores=_NS)
    scratch = [
        pltpu.VMEM((C * deg,), jnp.int32),
        pltpu.VMEM((C * deg + 16,), jnp.float32),
        pltpu.VMEM((C * deg, fw), jnp.float32),
        pltpu.VMEM((C, fw), jnp.float32),
    ]
    if fuse:
        scratch.append(pltpu.VMEM((C, fw), jnp.float32))
    scratch.append(pltpu.SemaphoreType.DMA)

    def body(*refs):
        if fuse:
            (x_hbm, cols_hbm, vals_hbm, sub_hbm, out_hbm,
             idx_v, vals_v, rows_v, out_v, sub_v, sem) = refs
        else:
            (x_hbm, cols_hbm, vals_hbm, out_hbm,
             idx_v, vals_v, rows_v, out_v, sem) = refs
        wid = lax.axis_index("s") * _NC + lax.axis_index("c")

        def chunk(t, carry):
            ci = wid * cpw + t
            base = ci * C
            pltpu.sync_copy(cols_hbm.at[pl.ds(base * deg, C * deg)], idx_v)
            pltpu.sync_copy(vals_hbm.at[pl.ds(base * deg, C * deg)],
                            vals_v.at[pl.ds(0, C * deg)])
            copies = []
            for g in range(G):
                copies.append(pltpu.async_copy(
                    x_hbm.at[idx_v.at[pl.ds(g * 128, 128)]],
                    rows_v.at[pl.ds(g * 128, 128)], sem))
            if fuse:
                pltpu.sync_copy(sub_hbm.at[pl.ds(base, C)], sub_v)
            for cp in copies:
                cp.wait()

            def row(r, carry2):
                e0 = r * deg
                wv = vals_v[pl.ds(e0, 16)]
                ws = [wv[d] for d in range(deg)]
                for f in range(fw // 16):
                    sl = pl.ds(f * 16, 16)
                    acc = ws[0] * rows_v[e0, sl]
                    for d in range(1, deg):
                        acc = acc + ws[d] * rows_v[e0 + d, sl]
                    if fuse:
                        acc = 2.0 * acc - sub_v[r, sl]
                    out_v[r, sl] = acc
                return carry2

            lax.fori_loop(0, C, row, 0)
            pltpu.sync_copy(out_v, out_hbm.at[pl.ds(base, C)])
            return carry

        lax.fori_loop(0, cpw, chunk, 0)

    args = (x, cols2d, vals) + ((sub,) if fuse else ())
    return pl.kernel(
        body,
        out_type=jax.ShapeDtypeStruct((n_pad, fw), jnp.float32),
        mesh=mesh,
        scratch_types=scratch,
        compiler_params=pltpu.CompilerParams(use_tc_tiling_on_sc=False),
    )(*args)


def _spmm(g, x, n, deg, sub=None):
    m, fw_in = x.shape
    if deg == 8:
        C = 64 if n >= 4096 else 16
    else:
        C = 128
    n_pad = _round_up(n, C * _NW)
    e_pad = n_pad * deg
    cols = g["cols"]
    vals = g["vals"]
    if e_pad > cols.shape[0]:
        cols = jnp.pad(cols, (0, e_pad - cols.shape[0]))
        vals = jnp.pad(vals, (0, e_pad - vals.shape[0]))
    cols1d = cols

    fw = _round_up(fw_in, 32)
    if fw != fw_in:
        x = jnp.pad(x, ((0, 0), (0, fw - fw_in)))
        if sub is not None:
            sub = jnp.pad(sub, ((0, 0), (0, fw - fw_in)))
    if sub is not None and sub.shape[0] != n_pad:
        sub = jnp.pad(sub, ((0, n_pad - sub.shape[0]), (0, 0)))

    outs = []
    for f0 in range(0, fw, 128):
        f1 = min(f0 + 128, fw)
        xs = x[:, f0:f1]
        ss = sub[:, f0:f1] if sub is not None else None
        outs.append(_spmm_sc(xs, cols1d, vals, ss, deg=deg, n_pad=n_pad,
                             c_rows=C, fuse=sub is not None))
    out = outs[0] if len(outs) == 1 else jnp.concatenate(outs, axis=1)
    return out[:n, :fw_in]


def _cheb(xm, Lg, W, b, B):
    M, bf = xm.shape
    fin = bf // B
    fo = W.shape[1]
    x0 = xm
    x1 = _spmm(Lg, x0, M, 8)
    xs = [x0, x1]
    for _ in range(2, _K):
        x2 = _spmm(Lg, x1, M, 8, sub=x0)
        xs.append(x2)
        x0, x1 = x1, x2
    Wk = W.reshape(fin, _K, fo)
    y = b
    for k in range(_K):
        y = y + xs[k].reshape(M * B, fin) @ Wk[:, k, :]
    return y.reshape(M, B * fo)


def _cheb_res_block(xm, Lg, blk, B):
    h = jax.nn.relu(_cheb(xm, Lg, blk["w1"], blk["b1"], B))
    h2 = _cheb(h, Lg, blk["w2"], blk["b2"], B)
    M = xm.shape[0]
    if "ws" in blk:
        sc = (xm.reshape(-1, xm.shape[1] // B) @ blk["ws"]).reshape(M, -1)
    else:
        sc = xm
    return jax.nn.relu(h2 + sc)


def _conv_block(x, pr, cfg):
    ci, co, k, s, p, res = cfg
    out = lax.conv_general_dilated(x, pr["w"], window_strides=s,
                                   padding=[(p, p), (p, p)],
                                   dimension_numbers=("NCHW", "OIHW", "NCHW"))
    out = out + pr["b"][None, :, None, None]
    out = pr["g"][None, :, None, None] * out + pr["be"][None, :, None, None]
    if res:
        out = out + x
    return jax.nn.relu(out)


def kernel(x, params, graphs):
    out = x
    for cfg, pr in zip(_CONV_CFG, params["convs"]):
        out = _conv_block(out, pr, cfg)
    out = out.reshape(out.shape[0], -1)
    out = out / (jnp.linalg.norm(out, axis=1, keepdims=True) + 1e-12)
    B = out.shape[0]
    h = jax.nn.relu(out @ params["fc_w"] + params["fc_b"])
    hm = h.reshape(B, _POOL[-1], _DECF[0]).transpose(1, 0, 2).reshape(_POOL[-1], B * _DECF[0])
    lap, ups, blks = graphs["L"], graphs["U"], params["blocks"]
    for lvl in range(4):
        g = ups[3 - lvl]
        n_out = _POOL[3 - lvl]
        hm = _spmm(g, hm, n_out, 3)
        hm = _cheb_res_block(hm, lap[3 - lvl], blks[lvl], B)
    hm = _cheb(hm, lap[0], params["last_w"], params["last_b"], B)
    M = hm.shape[0]
    return hm.reshape(M, B, 3).transpose(1, 0, 2).reshape(B, -1)


# speedup vs baseline: 9.4781x; 3.8295x over previous
import functools

import jax
import jax.numpy as jnp
import numpy as np
from jax import lax
from jax.experimental import pallas as pl
from jax.experimental.pallas import tpu as pltpu
from jax.experimental.pallas import tpu_sc as plsc

_CONV_CFG = [(1, 32, 3, (1, 1), 1, False), (32, 32, 3, (1, 1), 1, True), (32, 32, 3, (1, 1), 1, True), (32, 64, 3, (3, 1), 1, False), (64, 64, 3, (1, 1), 1, True), (64, 64, 3, (1, 1), 1, True), (64, 128, 3, (3, 3), 1, False), (128, 128, 3, (1, 1), 1, True), (128, 128, 3, (1, 1), 1, True), (128, 256, 3, (3, 2), 1, False), (256, 256, 3, (1, 1), 1, True), (256, 512, 3, (1, 1), 0, False), (512, 512, 1, (1, 1), 0, False)]
_POOL = [35709, 8928, 2232, 558, 140]
_K = 6
_DECF = [32, 16, 16, 16, 3]

_NC = 2
_NS = 16
_NW = _NC * _NS


def _round_up(a, b):
    return (a + b - 1) // b * b


@functools.partial(jax.jit, static_argnames=("deg", "n_pad", "c_rows", "fuse"))
def _spmm_sc(x, cols, vals, sub, *, deg, n_pad, c_rows, fuse):
    m, fw = x.shape
    C = c_rows
    E = C * deg
    assert E <= 128 or E % 128 == 0
    groups = [(g * 128, min(128, E - g * 128)) for g in range((E + 127) // 128)]
    assert n_pad % C == 0
    n_chunks = n_pad // C

    mesh = plsc.VectorSubcoreMesh(core_axis_name="c", subcore_axis_name="s",
                                  num_cores=_NC, num_subcores=_NS)
    scratch = [
        pltpu.VMEM((E,), jnp.int32),
        pltpu.VMEM((E + 16,), jnp.float32),
        pltpu.VMEM((E, fw), jnp.float32),
        pltpu.VMEM((C, fw), jnp.float32),
    ]
    if fuse:
        scratch.append(pltpu.VMEM((C, fw), jnp.float32))
    scratch.append(pltpu.SemaphoreType.DMA)

    def body(*refs):
        if fuse:
            (x_hbm, cols_hbm, vals_hbm, sub_hbm, out_hbm,
             idx_v, vals_v, rows_v, out_v, sub_v, sem) = refs
        else:
            (x_hbm, cols_hbm, vals_hbm, out_hbm,
             idx_v, vals_v, rows_v, out_v, sem) = refs
        wid = lax.axis_index("s") * _NC + lax.axis_index("c")
        my_chunks = (n_chunks - wid + _NW - 1) // _NW

        def chunk(t, carry):
            ci = t * _NW + wid
            base = ci * C
            pltpu.sync_copy(cols_hbm.at[pl.ds(base * deg, E)], idx_v)
            pltpu.sync_copy(vals_hbm.at[pl.ds(base * deg, E)],
                            vals_v.at[pl.ds(0, E)])
            copies = []
            for off, sz in groups:
                copies.append(pltpu.async_copy(
                    x_hbm.at[idx_v.at[pl.ds(off, sz)]],
                    rows_v.at[pl.ds(off, sz)], sem))
            if fuse:
                pltpu.sync_copy(sub_hbm.at[pl.ds(base, C)], sub_v)
            for cp in copies:
                cp.wait()

            def row(r, carry2):
                e0 = r * deg
                wv = vals_v[pl.ds(e0, 16)]
                ws = [wv[d] for d in range(deg)]
                for f in range(fw // 16):
                    sl = pl.ds(f * 16, 16)
                    acc = ws[0] * rows_v[e0, sl]
                    for d in range(1, deg):
                        acc = acc + ws[d] * rows_v[e0 + d, sl]
                    if fuse:
                        acc = 2.0 * acc - sub_v[r, sl]
                    out_v[r, sl] = acc
                return carry2

            lax.fori_loop(0, C, row, 0)
            pltpu.sync_copy(out_v, out_hbm.at[pl.ds(base, C)])
            return carry

        lax.fori_loop(0, my_chunks, chunk, 0)

    args = (x, cols, vals) + ((sub,) if fuse else ())
    return pl.kernel(
        body,
        out_type=jax.ShapeDtypeStruct((n_pad, fw), jnp.float32),
        mesh=mesh,
        scratch_types=scratch,
        compiler_params=pltpu.CompilerParams(use_tc_tiling_on_sc=False),
    )(*args)


def _spmm(g, x, n_pad, deg, sub=None):
    m, fw = x.shape
    assert fw % 32 == 0
    if deg == 8:
        C = 64 if n_pad >= 4096 else 16
    else:
        C = 128 if n_pad >= 8192 else 16
    assert n_pad % C == 0
    e_pad = n_pad * deg
    cols = g["cols"]
    vals = g["vals"]
    if e_pad > cols.shape[0]:
        cols = jnp.pad(cols, (0, e_pad - cols.shape[0]))
        vals = jnp.pad(vals, (0, e_pad - vals.shape[0]))

    outs = []
    for f0 in range(0, fw, 128):
        f1 = min(f0 + 128, fw)
        xs = x[:, f0:f1]
        ss = sub[:, f0:f1] if sub is not None else None
        outs.append(_spmm_sc(xs, cols, vals, ss, deg=deg, n_pad=n_pad,
                             c_rows=C, fuse=sub is not None))
    return outs[0] if len(outs) == 1 else jnp.concatenate(outs, axis=1)


@functools.partial(jax.jit, static_argnames=("relu", "res_mode", "blk_r"))
def _chebc_tc(xs, wp, bb, res, wsbd, *, relu, res_mode, blk_r):
    mp, wi = xs[0].shape
    wo = wp.shape[2]
    grid = ((mp + blk_r - 1) // blk_r,)

    def body(*refs):
        xrefs = refs[:_K]
        w_ref, b_ref = refs[_K], refs[_K + 1]
        o_ref = refs[-1]
        acc = b_ref[0][None, :]
        for k in range(_K):
            acc = acc + jnp.dot(xrefs[k][...], w_ref[k],
                                preferred_element_type=jnp.float32)
        if res_mode == 1:
            acc = acc + refs[_K + 2][...]
        elif res_mode == 2:
            acc = acc + jnp.dot(refs[_K + 2][...], refs[_K + 3][...],
                                preferred_element_type=jnp.float32)
        if relu:
            acc = jnp.maximum(acc, 0.0)
        o_ref[...] = acc

    in_specs = [pl.BlockSpec((blk_r, wi), lambda i: (i, 0)) for _ in range(_K)]
    in_specs.append(pl.BlockSpec((_K, wi, wo), lambda i: (0, 0, 0)))
    in_specs.append(pl.BlockSpec((1, wo), lambda i: (0, 0)))
    args = list(xs) + [wp, bb[None, :]]
    if res_mode >= 1:
        in_specs.append(pl.BlockSpec((blk_r, res.shape[1]), lambda i: (i, 0)))
        args.append(res)
    if res_mode == 2:
        in_specs.append(pl.BlockSpec(wsbd.shape, lambda i: (0, 0)))
        args.append(wsbd)

    return pl.pallas_call(
        body,
        grid=grid,
        in_specs=in_specs,
        out_specs=pl.BlockSpec((blk_r, wo), lambda i: (i, 0)),
        out_shape=jax.ShapeDtypeStruct((mp, wo), jnp.float32),
    )(*args)


def _bd(wk, b_sz, wi, wo):
    fin, fo = wk.shape
    out = jnp.zeros((wi, wo), jnp.float32)
    return out.at[:b_sz * fin, :b_sz * fo].set(jnp.kron(jnp.eye(b_sz), wk))


def _cheb(xm, Lg, W, b, B, fin, fo, relu, res=None, ws=None):
    mp, wi = xm.shape
    x0 = xm
    x1 = _spmm(Lg, x0, mp, 8)
    xs = [x0, x1]
    for _ in range(2, _K):
        x2 = _spmm(Lg, x1, mp, 8, sub=x0)
        xs.append(x2)
        x0, x1 = x1, x2
    wo = _round_up(B * fo, 32)
    wk3 = W.reshape(fin, _K, fo)
    wp = jnp.stack([_bd(wk3[:, k, :], B, wi, wo) for k in range(_K)], 0)
    bb = jnp.pad(jnp.tile(b, B), (0, wo - B * fo))
    res_mode = 0
    wsbd = None
    if res is not None:
        res_mode = 1
        if ws is not None:
            res_mode = 2
            wsbd = _bd(ws, B, res.shape[1], wo)
    blk_r = min(1024, mp)
    return _chebc_tc(tuple(xs), wp, bb, res, wsbd, relu=relu,
                     res_mode=res_mode, blk_r=blk_r)


def _cheb_res_block(xm, Lg, blk, B, fin, fo):
    h = _cheb(xm, Lg, blk["w1"], blk["b1"], B, fin, fo, relu=True)
    return _cheb(h, Lg, blk["w2"], blk["b2"], B, fo, fo, relu=True,
                 res=xm, ws=blk.get("ws"))


def _conv_block(x, pr, cfg):
    ci, co, k, s, p, res = cfg
    out = lax.conv_general_dilated(x, pr["w"], window_strides=s,
                                   padding=[(p, p), (p, p)],
                                   dimension_numbers=("NCHW", "OIHW", "NCHW"))
    out = out + pr["b"][None, :, None, None]
    out = pr["g"][None, :, None, None] * out + pr["be"][None, :, None, None]
    if res:
        out = out + x
    return jax.nn.relu(out)


def kernel(x, params, graphs):
    out = x
    for cfg, pr in zip(_CONV_CFG, params["convs"]):
        out = _conv_block(out, pr, cfg)
    out = out.reshape(out.shape[0], -1)
    out = out / (jnp.linalg.norm(out, axis=1, keepdims=True) + 1e-12)
    B = out.shape[0]
    h = jax.nn.relu(out @ params["fc_w"] + params["fc_b"])
    hm = h.reshape(B, _POOL[-1], _DECF[0]).transpose(1, 0, 2).reshape(_POOL[-1], B * _DECF[0])
    lap, ups, blks = graphs["L"], graphs["U"], params["blocks"]
    for lvl in range(4):
        mp = _round_up(_POOL[3 - lvl], 128)
        hm = _spmm(ups[3 - lvl], hm, mp, 3)
        hm = _cheb_res_block(hm, lap[3 - lvl], blks[lvl], B,
                             _DECF[lvl], _DECF[lvl + 1])
    hm = _cheb(hm, lap[0], params["last_w"], params["last_b"], B, 3, 3,
               relu=False)
    M = _POOL[0]
    return hm[:M, :B * 3].reshape(M, B, 3).transpose(1, 0, 2).reshape(B, -1)
